# baseline (speedup 1.0000x reference)
import functools

import jax
import jax.numpy as jnp
from jax import lax
from jax.experimental import pallas as pl
from jax.experimental.pallas import tpu as pltpu
import jax.experimental.pallas.tpu_sc as plsc

N = 320000
D = 128
S = 2048
NC = 2
NS = 16
NW = NC * NS
C = 80
NCHUNKS = 50
RPT = NCHUNKS * C
NSC = NW * RPT
NBUF = 5
PD = 3
NOUTER = NCHUNKS // NBUF
SROWS = S // NS
LANES = 16

R = 6400
B = (N - NSC) // R
W = 48


def _tile_body(nodes_hbm, batch2d_hbm, pout_hbm,
               rows_v, ids_v, zbuf_v, acc_sh, *sems):
    rsems = sems[:NBUF]
    ssems = sems[NBUF:]
    sid = lax.axis_index("s")
    cid = lax.axis_index("c")
    wid = cid * NS + sid
    row0 = wid * RPT

    def chunk_base(k):
        return pl.multiple_of(jnp.minimum(row0 + k * C, N - C), 8)

    def fetch(k, b):
        pltpu.async_copy(nodes_hbm.at[pl.ds(chunk_base(k), C)],
                         rows_v.at[b], rsems[b])

    def wait_fetch(k, b):
        pltpu.make_async_copy(
            nodes_hbm.at[pl.ds(chunk_base(k), C)],
            rows_v.at[b], rsems[b]).wait()

    def scatter(k, b):
        pltpu.async_copy(rows_v.at[b], acc_sh.at[ids_v.at[k]],
                         ssems[b], add=True)

    def wait_scatter(k, b):
        pltpu.make_async_copy(
            rows_v.at[b], acc_sh.at[ids_v.at[k]], ssems[b]).wait()

    pltpu.sync_copy(batch2d_hbm.at[wid], ids_v)
    for b in range(PD):
        fetch(jnp.int32(b), b)

    zz = jnp.zeros((LANES,), jnp.float32)

    def zero_row(i, carry):
        for j in range(D // LANES):
            zbuf_v[i, pl.ds(j * LANES, LANES)] = zz
        return carry

    lax.fori_loop(0, SROWS, zero_row, 0)
    pltpu.sync_copy(zbuf_v, acc_sh.at[pl.ds(sid * SROWS, SROWS)])
    plsc.subcore_barrier()

    def outer(k0, carry):
        for b in range(NBUF):
            k = k0 * NBUF + b
            wait_fetch(k, b)
            b2 = (b + PD) % NBUF
            if b in (0, 1):
                @pl.when(k0 > 0)
                def _():
                    wait_scatter(k - 2, b2)
            else:
                wait_scatter(k - 2, b2)
            scatter(k, b)
            fetch(k + PD, b2)
        return carry

    lax.fori_loop(0, NOUTER, outer, 0)

    for b in range(PD):
        wait_fetch(NCHUNKS + b, b)
    wait_scatter(NCHUNKS - 2, NBUF - 2)
    wait_scatter(NCHUNKS - 1, NBUF - 1)

    plsc.subcore_barrier()
    pltpu.sync_copy(acc_sh.at[pl.ds(sid * SROWS, SROWS)],
                    pout_hbm.at[cid].at[pl.ds(sid * SROWS, SROWS)])


@functools.partial(
    pl.kernel,
    out_type=jax.ShapeDtypeStruct((NC, S, D), jnp.float32),
    mesh=plsc.VectorSubcoreMesh(core_axis_name="c", subcore_axis_name="s"),
    scratch_types=[
        pltpu.VMEM((NBUF, C, D), jnp.float32),
        pltpu.VMEM((NCHUNKS, C), jnp.int32),
        pltpu.VMEM((SROWS, D), jnp.float32),
        pltpu.MemorySpace.VMEM_SHARED((S, D), jnp.float32),
    ] + [pltpu.SemaphoreType.DMA] * (2 * NBUF),
)
def _segment_sum_sc(nodes_hbm, batch2d_hbm, pout_hbm,
                    rows_v, ids_v, zbuf_v, acc_sh, *sems):
    _tile_body(nodes_hbm, batch2d_hbm, pout_hbm,
               rows_v, ids_v, zbuf_v, acc_sh, *sems)


def _tc_body(ids_ref, x_ref, o_ref):
    @pl.when(pl.program_id(0) == 0)
    def _():
        o_ref[...] = jnp.zeros((S, D), jnp.float32)

    idv = ids_ref[0]
    x = x_ref[...]
    first = jnp.min(idv)
    last = jnp.max(idv)
    wiota = lax.broadcasted_iota(jnp.int32, (W, R), 0)

    def cond(ws):
        return ws <= last

    def body(ws):
        lws = jnp.minimum(ws, S - W)
        oh = (idv + jnp.zeros((W, R), jnp.int32) == lws + wiota)
        win = jax.lax.dot_general(
            oh.astype(jnp.float32), x, (((1,), (0,)), ((), ())),
            preferred_element_type=jnp.float32)
        o_ref[pl.ds(lws, W), :] += win
        nxt = jnp.min(jnp.where(idv >= lws + W, idv, jnp.int32(S + W)))
        return nxt

    lax.while_loop(cond, body, first)


def _tc_partial(ids_tc, rows_tc):
    return pl.pallas_call(
        _tc_body,
        grid=(B,),
        in_specs=[
            pl.BlockSpec((1, 1, R), lambda i: (i, 0, 0)),
            pl.BlockSpec((R, D), lambda i: (NSC // R + i, 0)),
        ],
        out_specs=pl.BlockSpec((S, D), lambda i: (0, 0)),
        out_shape=jax.ShapeDtypeStruct((S, D), jnp.float32),
    )(ids_tc, rows_tc)


def _combine_body(p_ref, t_ref, o_ref):
    o_ref[...] = p_ref[0] + p_ref[1] + t_ref[...]


def _combine(partials, ptc):
    blk = 256
    return pl.pallas_call(
        _combine_body,
        grid=(S // blk,),
        in_specs=[
            pl.BlockSpec((NC, blk, D), lambda i: (0, i, 0)),
            pl.BlockSpec((blk, D), lambda i: (i, 0)),
        ],
        out_specs=pl.BlockSpec((blk, D), lambda i: (i, 0)),
        out_shape=jax.ShapeDtypeStruct((S, D), jnp.float32),
    )(partials, ptc)


def kernel(node_features, batch, ptr):
    partials = _segment_sum_sc(
        node_features, batch[:NSC].reshape(NW, NCHUNKS, C))
    ptc = _tc_partial(batch[NSC:].reshape(B, 1, R), node_features)
    return _combine(partials, ptc)
